# baseline (speedup 1.0000x reference)
import functools

import jax
import jax.numpy as jnp
from jax import lax
from jax.experimental import pallas as pl
from jax.experimental.pallas import tpu as pltpu
from jax.experimental.pallas import tpu_sc as plsc

NUM_GRAPHS = 1024

_C = 80
_GROWS = 32
_BM = _GROWS * _C


def _gate_body(h_ref, wgt_ref, bg_ref, g_ref):
    hb = h_ref[...]
    s = lax.dot_general(
        wgt_ref[...], hb, (((1,), (1,)), ((), ())),
        preferred_element_type=jnp.float32,
    )
    s = s + bg_ref[...]
    g_ref[0] = 1.0 / (1.0 + jnp.exp(-s))


def _gate_stage(h, W_g, b_g, block_base, nblocks):
    _, ndim = h.shape
    g3d = pl.pallas_call(
        _gate_body,
        grid=(nblocks,),
        in_specs=[
            pl.BlockSpec((_BM, ndim), lambda i: (i + block_base, 0)),
            pl.BlockSpec((1, ndim), lambda i: (0, 0)),
            pl.BlockSpec((1, 1), lambda i: (0, 0)),
        ],
        out_specs=pl.BlockSpec((1, 1, _BM), lambda i: (i, 0, 0)),
        out_shape=jax.ShapeDtypeStruct((nblocks, 1, _BM), jnp.float32),
    )(h, W_g.reshape(1, ndim), b_g.reshape(1, 1))
    return g3d.reshape(nblocks * _BM)


def _splat(vec, lane):
    lane_v = (jnp.full((16,), 0, jnp.int32) + lane)[:, None]
    return lax.gather(
        vec,
        lane_v,
        dimension_numbers=lax.GatherDimensionNumbers(
            offset_dims=(),
            collapsed_slice_dims=(0,),
            start_index_map=(0,),
        ),
        slice_sizes=(1,),
        mode=lax.GatherScatterMode.PROMISE_IN_BOUNDS,
    )


def _make_scatter(ndim, chunk_base, nchunk_part, g_row_base):
    rows_per_sub = NUM_GRAPHS // 16
    mesh = plsc.VectorSubcoreMesh(core_axis_name="c", subcore_axis_name="s")

    @functools.partial(
        pl.kernel,
        out_type=(
            jax.ShapeDtypeStruct((2, NUM_GRAPHS, ndim), jnp.float32),
            jax.ShapeDtypeStruct((2, NUM_GRAPHS), jnp.float32),
        ),
        mesh=mesh,
        scratch_types=[
            pltpu.VMEM((_C,), jnp.int32),
            pltpu.VMEM((_C,), jnp.int32),
            pltpu.VMEM((_C,), jnp.float32),
            pltpu.VMEM((_C,), jnp.float32),
            pltpu.VMEM((_C, ndim), jnp.float32),
            pltpu.VMEM((_C, ndim), jnp.float32),
            pltpu.VMEM((NUM_GRAPHS // 16,), jnp.float32),
            pltpu.VMEM_SHARED((NUM_GRAPHS, ndim), jnp.float32),
            pltpu.VMEM_SHARED((NUM_GRAPHS,), jnp.float32),
            pltpu.SemaphoreType.DMA,
            pltpu.SemaphoreType.DMA,
        ],
    )
    def _scatter(h_hbm, batch_hbm, g1d_hbm, zeros_hbm,
                 outh_hbm, outg_hbm,
                 idx0, idx1, g0, g1, rows0, rows1, gbuf, acc, acc_g,
                 sem0, sem1):
        cid = lax.axis_index("c")
        sid = lax.axis_index("s")
        wid = sid * 2 + cid

        pltpu.sync_copy(
            zeros_hbm.at[pl.ds(sid * rows_per_sub, rows_per_sub)],
            acc.at[pl.ds(sid * rows_per_sub, rows_per_sub)],
        )
        for i in range(rows_per_sub // 16):
            gbuf[pl.ds(i * 16, 16)] = jnp.zeros((16,), jnp.float32)
        pltpu.sync_copy(gbuf, acc_g.at[pl.ds(sid * rows_per_sub, rows_per_sub)])
        plsc.subcore_barrier()

        nt = (nchunk_part - wid + 31) // 32

        def srcs(t, idx_v, g_v, rows_v):
            c = chunk_base + wid + t * 32
            row0 = c * _C
            return (
                (batch_hbm.at[pl.ds(row0, _C)], idx_v),
                (g1d_hbm.at[pl.ds(row0 - g_row_base, _C)], g_v),
                (h_hbm.at[pl.ds(row0, _C)], rows_v),
            )

        def start_load(t, idx_v, g_v, rows_v, sem):
            for src, dst in srcs(t, idx_v, g_v, rows_v):
                pltpu.async_copy(src, dst, sem)

        def wait_load(t, idx_v, g_v, rows_v, sem):
            for src, dst in srcs(t, idx_v, g_v, rows_v):
                pltpu.make_async_copy(src, dst, sem).wait()

        def process(t, idx_v, g_v, rows_v, sem, idx_n, g_n, rows_n, sem_n):
            wait_load(t, idx_v, g_v, rows_v, sem)

            @pl.when(t + 1 < nt)
            def _():
                start_load(t + 1, idx_n, g_n, rows_n, sem_n)

            @plsc.parallel_loop(0, _C, unroll=4)
            def scale_row(r):
                grp = (r // 16) * 16
                gvec = g_v[pl.ds(grp, 16)]
                gs = _splat(gvec, r - grp)
                for j in range(ndim // 16):
                    rows_v[r, pl.ds(j * 16, 16)] = (
                        rows_v[r, pl.ds(j * 16, 16)] * gs
                    )

            pltpu.sync_copy(rows_v, acc.at[idx_v], add=True)
            pltpu.sync_copy(g_v, acc_g.at[idx_v], add=True)

        start_load(0, idx0, g0, rows0, sem0)

        def body(t, carry):
            @pl.when(t % 2 == 0)
            def _():
                process(t, idx0, g0, rows0, sem0, idx1, g1, rows1, sem1)

            @pl.when(t % 2 == 1)
            def _():
                process(t, idx1, g1, rows1, sem1, idx0, g0, rows0, sem0)

            return carry

        lax.fori_loop(0, nt, body, 0)
        plsc.subcore_barrier()

        pltpu.sync_copy(
            acc.at[pl.ds(sid * rows_per_sub, rows_per_sub)],
            outh_hbm.at[cid, pl.ds(sid * rows_per_sub, rows_per_sub)],
        )
        pltpu.sync_copy(
            acc_g.at[pl.ds(sid * rows_per_sub, rows_per_sub)], gbuf
        )
        pltpu.sync_copy(
            gbuf, outg_hbm.at[cid, pl.ds(sid * rows_per_sub, rows_per_sub)]
        )

    return _scatter


def _final(phs, pgs, W_f, b_f):
    gdim = W_f.shape[1]
    nparts = len(phs)

    def _final_body(*refs):
        ph_refs = refs[:nparts]
        pg_refs = refs[nparts:2 * nparts]
        wf_ref, bf_ref, o_ref = refs[2 * nparts:]
        pooled = ph_refs[0][0] + ph_refs[0][1]
        gsum = pg_refs[0][0:1, :] + pg_refs[0][1:2, :]
        for p in range(1, nparts):
            pooled = pooled + ph_refs[p][0] + ph_refs[p][1]
            gsum = gsum + pg_refs[p][0:1, :] + pg_refs[p][1:2, :]
        bias = lax.dot_general(
            gsum, bf_ref[...], (((0,), (0,)), ((), ())),
            preferred_element_type=jnp.float32,
        )
        o_ref[...] = (
            jnp.dot(pooled, wf_ref[...], preferred_element_type=jnp.float32)
            + bias
        )

    return pl.pallas_call(
        _final_body,
        out_shape=jax.ShapeDtypeStruct((NUM_GRAPHS, gdim), jnp.float32),
    )(*phs, *pgs, W_f, b_f.reshape(1, gdim))


_NPARTS = 4


def kernel(h, batch, W_f, b_f, W_g, b_g):
    n, ndim = h.shape
    nchunk = n // _C
    nblocks_all = pl.cdiv(n, _BM)

    idx = batch.astype(jnp.int32)
    zeros = jnp.zeros((NUM_GRAPHS, ndim), jnp.float32)

    base = nchunk // _NPARTS
    rem = nchunk - base * _NPARTS
    bounds = [0]
    for p in range(_NPARTS):
        bounds.append(bounds[-1] + base + (1 if p < rem else 0))

    phs, pgs = [], []
    for p in range(_NPARTS):
        cb, ce = bounds[p], bounds[p + 1]
        b_lo = (cb * _C) // _BM
        b_hi = min(pl.cdiv(ce * _C, _BM), nblocks_all)
        g_p = _gate_stage(h, W_g, b_g, b_lo, b_hi - b_lo)
        ph, pg = _make_scatter(ndim, cb, ce - cb, b_lo * _BM)(
            h, idx, g_p, zeros
        )
        phs.append(ph)
        pgs.append(pg)
    return _final(phs, pgs, W_f, b_f)
